# baseline (speedup 1.0000x reference)
import functools

import jax
import jax.numpy as jnp
from jax import lax
from jax.experimental import pallas as pl
from jax.experimental.pallas import tpu as pltpu
from jax.experimental.pallas import tpu_sc as plsc

_NUM_CORES = 2
_NUM_SUBCORES = 16
_NUM_WORKERS = _NUM_CORES * _NUM_SUBCORES

_EMBED_DIM = 64
_BATCH = 16384
_HIST = 50
_TOTAL = _BATCH * _HIST

_CHUNK = 128
_K = 2
_PER_WORKER = _TOTAL // _NUM_WORKERS
_NUM_CHUNKS = _PER_WORKER // _CHUNK
_NUM_BLOCKS = _NUM_CHUNKS // _K
_BT = _BATCH // _CHUNK


def _make_gather():
    mesh = plsc.VectorSubcoreMesh(
        core_axis_name="c",
        subcore_axis_name="s",
        num_cores=_NUM_CORES,
        num_subcores=_NUM_SUBCORES,
    )

    @functools.partial(
        pl.kernel,
        out_type=jax.ShapeDtypeStruct(
            (_HIST, _EMBED_DIM // 8, _BT, 8, _CHUNK), jnp.float32
        ),
        mesh=mesh,
        scratch_types=[
            pltpu.VMEM((_NUM_CHUNKS, _CHUNK), jnp.int32),
            pltpu.VMEM((2, _K, _CHUNK, _EMBED_DIM), jnp.float32),
            pltpu.VMEM((2, _K, _EMBED_DIM // 8, 8, _CHUNK), jnp.float32),
            pltpu.SemaphoreType.DMA,
            pltpu.SemaphoreType.DMA,
        ],
        compiler_params=pltpu.CompilerParams(
            use_tc_tiling_on_sc=False, needs_layout_passes=False
        ),
    )
    def gather_kernel(idx_hbm, table_hbm, out_hbm, idx_v, rows_v, blk_v, gsem, ssem):
        wid = lax.axis_index("s") * _NUM_CORES + lax.axis_index("c")
        pltpu.sync_copy(idx_hbm.at[wid], idx_v)

        col_iota = lax.iota(jnp.int32, 16)

        def fire_gathers(t, grp):
            for b in range(_K):
                pltpu.async_copy(
                    table_hbm.at[idx_v.at[t * _K + b]], rows_v.at[grp, b], gsem
                )

        def drain(sem, is_store):
            if is_store:
                pltpu.make_async_copy(blk_v.at[0, 0], out_hbm.at[0, :, 0], sem).wait()
            else:
                pltpu.make_async_copy(
                    table_hbm.at[idx_v.at[0]], rows_v.at[0, 0], sem
                ).wait()

        def transpose_chunk(grp, b):
            src = rows_v.at[grp, b]
            dst = blk_v.at[grp, b]
            for c0 in range(0, _CHUNK, 16):
                row_idx = col_iota + c0
                for d in range(_EMBED_DIM):
                    vals = plsc.load_gather(
                        src, [row_idx, jnp.full((16,), d, jnp.int32)]
                    )
                    dst[d // 8, d % 8, pl.ds(c0, 16)] = vals

        fire_gathers(0, 0)

        def body(t, carry):
            grp = lax.rem(t, 2)
            for _ in range(_K):
                drain(gsem, is_store=False)

            @pl.when(t >= 1)
            def _():
                for _ in range(_K):
                    drain(ssem, is_store=True)

            @pl.when(t + 1 < _NUM_BLOCKS)
            def _():
                fire_gathers(t + 1, 1 - grp)

            for b in range(_K):
                transpose_chunk(grp, b)
                f0 = (wid * _NUM_CHUNKS + t * _K + b) * _CHUNK
                h = f0 // _BATCH
                bt = (f0 % _BATCH) // _CHUNK
                pltpu.async_copy(blk_v.at[grp, b], out_hbm.at[h, :, bt], ssem)
            return carry

        lax.fori_loop(0, _NUM_BLOCKS, body, 0, unroll=False)
        for _ in range(_K):
            drain(ssem, is_store=True)

    return gather_kernel


_gather = _make_gather()


def kernel(words, emb_weight):
    idx = words.T.reshape(_NUM_WORKERS, _NUM_CHUNKS, _CHUNK).astype(jnp.int32)
    out5 = _gather(idx, emb_weight)
    return out5.transpose(2, 4, 0, 1, 3).reshape(_BATCH, _HIST, _EMBED_DIM)


# speedup vs baseline: 1.4965x; 1.4965x over previous
import functools

import jax
import jax.numpy as jnp
from jax import lax
from jax.experimental import pallas as pl
from jax.experimental.pallas import tpu as pltpu
from jax.experimental.pallas import tpu_sc as plsc

_NUM_CORES = 2
_NUM_SUBCORES = 16
_NUM_WORKERS = _NUM_CORES * _NUM_SUBCORES

_EMBED_DIM = 64
_BATCH = 16384
_HIST = 50
_TOTAL = _BATCH * _HIST

_CHUNK = 128
_K = 2
_PER_WORKER = _TOTAL // _NUM_WORKERS
_NUM_CHUNKS = _PER_WORKER // _CHUNK
_NUM_BLOCKS = _NUM_CHUNKS // _K
_BT = _BATCH // _CHUNK


def _make_gather():
    mesh = plsc.VectorSubcoreMesh(
        core_axis_name="c",
        subcore_axis_name="s",
        num_cores=_NUM_CORES,
        num_subcores=_NUM_SUBCORES,
    )

    @functools.partial(
        pl.kernel,
        out_type=jax.ShapeDtypeStruct(
            (_HIST, _EMBED_DIM // 8, _BT, 8, _CHUNK), jnp.float32
        ),
        mesh=mesh,
        scratch_types=[
            pltpu.VMEM((_NUM_CHUNKS, _CHUNK), jnp.int32),
            pltpu.VMEM((2, _K, _CHUNK, _EMBED_DIM), jnp.float32),
            pltpu.VMEM((2, _K, _EMBED_DIM // 8, 8, _CHUNK), jnp.float32),
            pltpu.SemaphoreType.DMA,
            pltpu.SemaphoreType.DMA,
        ],
        compiler_params=pltpu.CompilerParams(
            use_tc_tiling_on_sc=False, needs_layout_passes=False
        ),
    )
    def gather_kernel(idx_hbm, table_hbm, out_hbm, idx_v, rows_v, blk_v, gsem, ssem):
        wid = lax.axis_index("s") * _NUM_CORES + lax.axis_index("c")
        pltpu.sync_copy(idx_hbm.at[wid], idx_v)

        col_iota = lax.iota(jnp.int32, 16)

        def fire_gathers(t, grp):
            for b in range(_K):
                pltpu.async_copy(
                    table_hbm.at[idx_v.at[t * _K + b]], rows_v.at[grp, b], gsem
                )

        def drain(sem, is_store):
            if is_store:
                pltpu.make_async_copy(blk_v.at[0, 0], out_hbm.at[0, :, 0], sem).wait()
            else:
                pltpu.make_async_copy(
                    table_hbm.at[idx_v.at[0]], rows_v.at[0, 0], sem
                ).wait()

        def transpose_chunk(grp, b):
            src = rows_v.at[grp, b]
            dst = blk_v.at[grp, b]
            for c0 in range(0, _CHUNK, 16):
                row_idx = col_iota + c0
                for d0 in range(0, _EMBED_DIM, 16):
                    vals = [
                        plsc.load_gather(
                            src, [row_idx, jnp.full((16,), d0 + i, jnp.int32)]
                        )
                        for i in range(16)
                    ]
                    for i in range(16):
                        d = d0 + i
                        dst[d // 8, d % 8, pl.ds(c0, 16)] = vals[i]

        fire_gathers(0, 0)

        def body(t, carry):
            grp = lax.rem(t, 2)
            for _ in range(_K):
                drain(gsem, is_store=False)

            @pl.when(t >= 1)
            def _():
                for _ in range(_K):
                    drain(ssem, is_store=True)

            @pl.when(t + 1 < _NUM_BLOCKS)
            def _():
                fire_gathers(t + 1, 1 - grp)

            for b in range(_K):
                transpose_chunk(grp, b)
                f0 = (wid * _NUM_CHUNKS + t * _K + b) * _CHUNK
                h = f0 // _BATCH
                bt = (f0 % _BATCH) // _CHUNK
                pltpu.async_copy(blk_v.at[grp, b], out_hbm.at[h, :, bt], ssem)
            return carry

        lax.fori_loop(0, _NUM_BLOCKS, body, 0, unroll=False)
        for _ in range(_K):
            drain(ssem, is_store=True)

    return gather_kernel


_gather = _make_gather()


def kernel(words, emb_weight):
    idx = words.T.reshape(_NUM_WORKERS, _NUM_CHUNKS, _CHUNK).astype(jnp.int32)
    out5 = _gather(idx, emb_weight)
    return out5.transpose(2, 4, 0, 1, 3).reshape(_BATCH, _HIST, _EMBED_DIM)


# speedup vs baseline: 2.4662x; 1.6480x over previous
import functools

import jax
import jax.numpy as jnp
from jax import lax
from jax.experimental import pallas as pl
from jax.experimental.pallas import tpu as pltpu
from jax.experimental.pallas import tpu_sc as plsc

_NUM_CORES = 2
_NUM_SUBCORES = 16
_NUM_WORKERS = _NUM_CORES * _NUM_SUBCORES

_EMBED_DIM = 64
_BATCH = 16384
_HIST = 50
_TOTAL = _BATCH * _HIST

_CHUNK = 128
_K = 2
_PER_WORKER = _TOTAL // _NUM_WORKERS
_NUM_CHUNKS = _PER_WORKER // _CHUNK
_NUM_BLOCKS = _NUM_CHUNKS // _K
_BT = _BATCH // _CHUNK


def _make_gather():
    mesh = plsc.VectorSubcoreMesh(
        core_axis_name="c",
        subcore_axis_name="s",
        num_cores=_NUM_CORES,
        num_subcores=_NUM_SUBCORES,
    )

    @functools.partial(
        pl.kernel,
        out_type=jax.ShapeDtypeStruct(
            (_HIST, _EMBED_DIM // 8, _BT, 8, _CHUNK), jnp.float32
        ),
        mesh=mesh,
        scratch_types=[
            pltpu.VMEM((_NUM_CHUNKS, _CHUNK), jnp.int32),
            pltpu.VMEM((2, _K, _CHUNK, _EMBED_DIM), jnp.float32),
            pltpu.VMEM((2, _K, _EMBED_DIM, _CHUNK), jnp.float32),
            pltpu.SemaphoreType.DMA,
            pltpu.SemaphoreType.DMA,
        ],
        compiler_params=pltpu.CompilerParams(
            use_tc_tiling_on_sc=False, needs_layout_passes=False
        ),
    )
    def gather_kernel(idx_hbm, table_hbm, out_hbm, idx_v, rows_v, blk_v, gsem, ssem):
        wid = lax.axis_index("s") * _NUM_CORES + lax.axis_index("c")
        pltpu.sync_copy(idx_hbm.at[wid], idx_v)

        col_iota = lax.iota(jnp.int32, 16)

        def fire_gathers(t, grp):
            for b in range(_K):
                pltpu.async_copy(
                    table_hbm.at[idx_v.at[t * _K + b]], rows_v.at[grp, b], gsem
                )

        def drain(sem, is_store):
            if is_store:
                pltpu.make_async_copy(
                    blk_v.at[0, 0, pl.ds(0, 8)], out_hbm.at[0, 0, 0], sem
                ).wait()
            else:
                pltpu.make_async_copy(
                    table_hbm.at[idx_v.at[0]], rows_v.at[0, 0], sem
                ).wait()

        rot = [jnp.bitwise_and(col_iota + k, 15) for k in range(16)]

        def transpose_chunk(grp, b):
            src = rows_v.at[grp, b]
            dst = blk_v.at[grp, b]

            def cbody(ci, car):
                row_idx = col_iota + ci * 16
                for d0 in range(0, _EMBED_DIM, 16):
                    for k0 in range(0, 16, 8):
                        d_idx = [rot[k0 + k] + d0 for k in range(8)]
                        vals = [
                            plsc.load_gather(src, [row_idx, d_idx[k]])
                            for k in range(8)
                        ]
                        for k in range(8):
                            plsc.store_scatter(dst, [d_idx[k], row_idx], vals[k])
                return car

            lax.fori_loop(0, _CHUNK // 16, cbody, 0, unroll=False)

        fire_gathers(0, 0)

        def body(t, carry):
            grp = lax.rem(t, 2)
            for _ in range(_K):
                drain(gsem, is_store=False)

            @pl.when(t >= 1)
            def _():
                for _ in range(_K * (_EMBED_DIM // 8)):
                    drain(ssem, is_store=True)

            @pl.when(t + 1 < _NUM_BLOCKS)
            def _():
                fire_gathers(t + 1, 1 - grp)

            for b in range(_K):
                transpose_chunk(grp, b)
                f0 = (wid * _NUM_CHUNKS + t * _K + b) * _CHUNK
                h = f0 // _BATCH
                bt = (f0 % _BATCH) // _CHUNK
                for dt in range(_EMBED_DIM // 8):
                    pltpu.async_copy(
                        blk_v.at[grp, b, pl.ds(dt * 8, 8)],
                        out_hbm.at[h, dt, bt],
                        ssem,
                    )
            return carry

        lax.fori_loop(0, _NUM_BLOCKS, body, 0, unroll=False)
        for _ in range(_K * (_EMBED_DIM // 8)):
            drain(ssem, is_store=True)

    return gather_kernel


_gather = _make_gather()


def kernel(words, emb_weight):
    idx = words.T.reshape(_NUM_WORKERS, _NUM_CHUNKS, _CHUNK).astype(jnp.int32)
    out5 = _gather(idx, emb_weight)
    return out5.transpose(2, 4, 0, 1, 3).reshape(_BATCH, _HIST, _EMBED_DIM)


# speedup vs baseline: 3.6654x; 1.4863x over previous
import functools

import jax
import jax.numpy as jnp
from jax import lax
from jax.experimental import pallas as pl
from jax.experimental.pallas import tpu as pltpu
from jax.experimental.pallas import tpu_sc as plsc

_NUM_CORES = 2
_NUM_SUBCORES = 16
_NUM_WORKERS = _NUM_CORES * _NUM_SUBCORES

_EMBED_DIM = 64
_BATCH = 16384
_HIST = 50
_TOTAL = _BATCH * _HIST

_CHUNK = 128
_K = 2
_PER_WORKER = _TOTAL // _NUM_WORKERS
_NUM_CHUNKS = _PER_WORKER // _CHUNK
_NUM_BLOCKS = _NUM_CHUNKS // _K
_BT = _BATCH // _CHUNK


def _make_gather():
    mesh = plsc.VectorSubcoreMesh(
        core_axis_name="c",
        subcore_axis_name="s",
        num_cores=_NUM_CORES,
        num_subcores=_NUM_SUBCORES,
    )

    @functools.partial(
        pl.kernel,
        out_type=jax.ShapeDtypeStruct(
            (_HIST, _EMBED_DIM // 8, _BT, 8, _CHUNK), jnp.float32
        ),
        mesh=mesh,
        scratch_types=[
            pltpu.VMEM((_NUM_CHUNKS, _CHUNK), jnp.int32),
            pltpu.VMEM((2, _K, _CHUNK, _EMBED_DIM), jnp.float32),
            pltpu.VMEM((2, _K, _EMBED_DIM, _CHUNK), jnp.float32),
            pltpu.SemaphoreType.DMA,
            pltpu.SemaphoreType.DMA,
        ],
        compiler_params=pltpu.CompilerParams(
            use_tc_tiling_on_sc=False, needs_layout_passes=False
        ),
    )
    def gather_kernel(idx_hbm, table_hbm, out_hbm, idx_v, rows_v, blk_v, gsem, ssem):
        wid = lax.axis_index("s") * _NUM_CORES + lax.axis_index("c")
        pltpu.sync_copy(idx_hbm.at[wid], idx_v)

        col_iota = lax.iota(jnp.int32, 16)

        def fire_gathers(t, grp):
            for b in range(_K):
                pltpu.async_copy(
                    table_hbm.at[idx_v.at[t * _K + b]], rows_v.at[grp, b], gsem
                )

        def drain(sem, is_store):
            if is_store:
                pltpu.make_async_copy(
                    blk_v.at[0, 0, pl.ds(0, 8)], out_hbm.at[0, 0, 0], sem
                ).wait()
            else:
                pltpu.make_async_copy(
                    table_hbm.at[idx_v.at[0]], rows_v.at[0, 0], sem
                ).wait()

        rot = [jnp.bitwise_and(col_iota + k, 15) for k in range(16)]

        def transpose_chunk(grp, b):
            src = rows_v.at[grp, b]
            dst = blk_v.at[grp, b]

            def cbody(ci, car):
                row_idx = col_iota + ci * 16
                for d0 in range(0, _EMBED_DIM, 16):
                    for k0 in range(0, 16, 8):
                        d_idx = [rot[k0 + k] + d0 for k in range(8)]
                        vals = [
                            plsc.load_gather(src, [row_idx, d_idx[k]])
                            for k in range(8)
                        ]
                        for k in range(8):
                            plsc.store_scatter(dst, [d_idx[k], row_idx], vals[k])
                return car

            lax.fori_loop(0, _CHUNK // 16, cbody, 0, unroll=False)

        fire_gathers(0, 0)

        def body(t, carry):
            grp = lax.rem(t, 2)
            for _ in range(_K):
                drain(gsem, is_store=False)

            @pl.when(t >= 1)
            def _():
                for _ in range(_K * (_EMBED_DIM // 8)):
                    drain(ssem, is_store=True)

            @pl.when(t + 1 < _NUM_BLOCKS)
            def _():
                fire_gathers(t + 1, 1 - grp)

            for b in range(_K):
                transpose_chunk(grp, b)
                f0 = (wid * _NUM_CHUNKS + t * _K + b) * _CHUNK
                h = f0 // _BATCH
                bt = (f0 % _BATCH) // _CHUNK
                for dt in range(_EMBED_DIM // 8):
                    pltpu.async_copy(
                        blk_v.at[grp, b, pl.ds(dt * 8, 8)],
                        out_hbm.at[h, dt, bt],
                        ssem,
                    )
            return carry

        lax.fori_loop(0, _NUM_BLOCKS, body, 0, unroll=False)
        for _ in range(_K * (_EMBED_DIM // 8)):
            drain(ssem, is_store=True)

    return gather_kernel


_gather = _make_gather()

_VOCAB = 1000000
_FULL_UNITS = _VOCAB // _CHUNK
_UNITS = _FULL_UNITS + 1
_UPW = (_UNITS + _NUM_WORKERS - 1) // _NUM_WORKERS


def _make_linearize():
    mesh = plsc.VectorSubcoreMesh(
        core_axis_name="c",
        subcore_axis_name="s",
        num_cores=_NUM_CORES,
        num_subcores=_NUM_SUBCORES,
    )

    @functools.partial(
        pl.kernel,
        out_type=jax.ShapeDtypeStruct((_VOCAB * _EMBED_DIM,), jnp.float32),
        mesh=mesh,
        scratch_types=[
            pltpu.VMEM((_EMBED_DIM, _CHUNK), jnp.float32),
            pltpu.VMEM((_EMBED_DIM, _CHUNK), jnp.float32),
            pltpu.VMEM((_CHUNK * _EMBED_DIM,), jnp.float32),
            pltpu.VMEM((_CHUNK * _EMBED_DIM,), jnp.float32),
            pltpu.SemaphoreType.DMA,
            pltpu.SemaphoreType.DMA,
        ],
        compiler_params=pltpu.CompilerParams(
            use_tc_tiling_on_sc=True, needs_layout_passes=False
        ),
    )
    def lin_kernel(tt_hbm, aux_hbm, out_hbm, in0_v, in1_v, ob0_v, ob1_v, gsem, ssem):
        ins = (in0_v, in1_v)
        obs = (ob0_v, ob1_v)
        wid = lax.axis_index("s") * _NUM_CORES + lax.axis_index("c")
        col_iota = lax.iota(jnp.int32, 16)
        rot = [jnp.bitwise_and(col_iota + k, 15) for k in range(16)]

        def unit_id(i):
            return i * _NUM_WORKERS + wid

        def fire_gather(i, gp):
            u = unit_id(i)

            @pl.when(u < _FULL_UNITS)
            def _():
                pltpu.async_copy(
                    tt_hbm.at[:, pl.ds(u * _CHUNK, _CHUNK)], ins[gp], gsem
                )

            @pl.when(u == _FULL_UNITS)
            def _():
                pltpu.async_copy(aux_hbm, ins[gp], gsem)

        fbase = [rot[k] * _EMBED_DIM + col_iota for k in range(16)]

        def transpose_unit(gp):
            src = ins[gp]
            dst = obs[gp]

            def jbody(ci, car):
                j0 = ci * 16
                for i0 in range(0, _EMBED_DIM, 16):
                    for k0 in range(0, 16, 8):
                        c_idx = [rot[k0 + k] + j0 for k in range(8)]
                        vals = [
                            plsc.load_gather(src, [i0 + col_iota, c_idx[k]])
                            for k in range(8)
                        ]
                        for k in range(8):
                            plsc.store_scatter(
                                dst,
                                [fbase[k0 + k] + (j0 * _EMBED_DIM + i0)],
                                vals[k],
                            )
                return car

            lax.fori_loop(0, _CHUNK // 16, jbody, 0, unroll=False)

        fire_gather(0, 0)

        def half_body(i, gp):
            u = unit_id(i)
            valid = u < _UNITS

            @pl.when(valid)
            def _():
                pltpu.make_async_copy(
                    tt_hbm.at[:, pl.ds(0, _CHUNK)], ins[0], gsem
                ).wait()

            @pl.when((i >= 1) & (unit_id(i - 1) < _UNITS))
            def _():
                pltpu.make_async_copy(
                    obs[0], out_hbm.at[pl.ds(0, _CHUNK * _EMBED_DIM)], ssem
                ).wait()

            @pl.when(unit_id(i + 1) < _UNITS)
            def _():
                fire_gather(i + 1, 1 - gp)

            @pl.when(valid)
            def _():
                transpose_unit(gp)
                f0 = jnp.where(
                    u == _FULL_UNITS,
                    (_VOCAB - _CHUNK) * _EMBED_DIM,
                    u * _CHUNK * _EMBED_DIM,
                )
                pltpu.async_copy(
                    obs[gp], out_hbm.at[pl.ds(f0, _CHUNK * _EMBED_DIM)], ssem
                )

        def body(t, carry):
            half_body(2 * t, 0)
            half_body(2 * t + 1, 1)
            return carry

        lax.fori_loop(0, _UPW // 2, body, 0, unroll=False)
        half_body(_UPW - 1, 0)

        @pl.when(unit_id(_UPW - 1) < _UNITS)
        def _():
            pltpu.make_async_copy(
                obs[0], out_hbm.at[pl.ds(0, _CHUNK * _EMBED_DIM)], ssem
            ).wait()

    return lin_kernel


_linearize = _make_linearize()


def kernel(words, emb_weight):
    idx = words.T.reshape(_NUM_WORKERS, _NUM_CHUNKS, _CHUNK).astype(jnp.int32)
    tt = emb_weight.T
    aux = lax.slice(tt, (0, _VOCAB - _CHUNK), (_EMBED_DIM, _VOCAB))
    table = _linearize(tt, aux).reshape(_VOCAB, _EMBED_DIM)
    out5 = _gather(idx, table)
    return out5.transpose(2, 4, 0, 1, 3).reshape(_BATCH, _HIST, _EMBED_DIM)


# speedup vs baseline: 3.9286x; 1.0718x over previous
import functools

import jax
import jax.numpy as jnp
from jax import lax
from jax.experimental import pallas as pl
from jax.experimental.pallas import tpu as pltpu
from jax.experimental.pallas import tpu_sc as plsc

_NUM_CORES = 2
_NUM_SUBCORES = 16
_NUM_WORKERS = _NUM_CORES * _NUM_SUBCORES

_EMBED_DIM = 64
_BATCH = 16384
_HIST = 50
_TOTAL = _BATCH * _HIST

_CHUNK = 128
_K = 2
_PER_WORKER = _TOTAL // _NUM_WORKERS
_NUM_CHUNKS = _PER_WORKER // _CHUNK
_NUM_BLOCKS = _NUM_CHUNKS // _K
_BT = _BATCH // _CHUNK


def _make_gather():
    mesh = plsc.VectorSubcoreMesh(
        core_axis_name="c",
        subcore_axis_name="s",
        num_cores=_NUM_CORES,
        num_subcores=_NUM_SUBCORES,
    )

    @functools.partial(
        pl.kernel,
        out_type=jax.ShapeDtypeStruct((_TOTAL * _EMBED_DIM,), jnp.float32),
        mesh=mesh,
        scratch_types=[
            pltpu.VMEM((_NUM_CHUNKS, _CHUNK), jnp.int32),
            pltpu.VMEM((2, _K, _CHUNK, _EMBED_DIM), jnp.float32),
            pltpu.VMEM((2, _K, _EMBED_DIM * _CHUNK), jnp.float32),
            pltpu.SemaphoreType.DMA,
            pltpu.SemaphoreType.DMA,
        ],
        compiler_params=pltpu.CompilerParams(
            use_tc_tiling_on_sc=False, needs_layout_passes=False
        ),
    )
    def gather_kernel(idx_hbm, table_hbm, out_hbm, idx_v, rows_v, blk_v, gsem, ssem):
        wid = lax.axis_index("s") * _NUM_CORES + lax.axis_index("c")
        pltpu.sync_copy(idx_hbm.at[wid], idx_v)

        col_iota = lax.iota(jnp.int32, 16)

        def fire_gathers(t, grp):
            for b in range(_K):
                pltpu.async_copy(
                    table_hbm.at[idx_v.at[t * _K + b]], rows_v.at[grp, b], gsem
                )

        def drain(sem, is_store):
            if is_store:
                pltpu.make_async_copy(
                    blk_v.at[0, 0, pl.ds(0, 8 * _CHUNK)],
                    out_hbm.at[pl.ds(0, 8 * _CHUNK)],
                    sem,
                ).wait()
            else:
                pltpu.make_async_copy(
                    table_hbm.at[idx_v.at[0]], rows_v.at[0, 0], sem
                ).wait()

        rot = [jnp.bitwise_and(col_iota + k, 15) for k in range(16)]
        sbase = [rot[k] * _CHUNK + col_iota for k in range(16)]

        def transpose_chunk(grp, b):
            src = rows_v.at[grp, b]
            dst = blk_v.at[grp, b]

            def cbody(ci, car):
                c0 = ci * 16
                row_idx = col_iota + c0
                for d0 in range(0, _EMBED_DIM, 16):
                    for k0 in range(0, 16, 8):
                        d_idx = [rot[k0 + k] + d0 for k in range(8)]
                        vals = [
                            plsc.load_gather(src, [row_idx, d_idx[k]])
                            for k in range(8)
                        ]
                        for k in range(8):
                            plsc.store_scatter(
                                dst,
                                [sbase[k0 + k] + (d0 * _CHUNK + c0)],
                                vals[k],
                            )
                return car

            lax.fori_loop(0, _CHUNK // 16, cbody, 0, unroll=False)

        fire_gathers(0, 0)

        def body(t, carry):
            grp = lax.rem(t, 2)
            for _ in range(_K):
                drain(gsem, is_store=False)

            @pl.when(t >= 1)
            def _():
                for _ in range(_K * (_EMBED_DIM // 8)):
                    drain(ssem, is_store=True)

            @pl.when(t + 1 < _NUM_BLOCKS)
            def _():
                fire_gathers(t + 1, 1 - grp)

            for b in range(_K):
                transpose_chunk(grp, b)
                f0 = (wid * _NUM_CHUNKS + t * _K + b) * _CHUNK
                h = f0 // _BATCH
                bt = (f0 % _BATCH) // _CHUNK
                for dt in range(_EMBED_DIM // 8):
                    tile0 = ((h * (_EMBED_DIM // 8) + dt) * _BT + bt) * (8 * _CHUNK)
                    pltpu.async_copy(
                        blk_v.at[grp, b, pl.ds(dt * 8 * _CHUNK, 8 * _CHUNK)],
                        out_hbm.at[pl.ds(tile0, 8 * _CHUNK)],
                        ssem,
                    )
            return carry

        lax.fori_loop(0, _NUM_BLOCKS, body, 0, unroll=False)
        for _ in range(_K * (_EMBED_DIM // 8)):
            drain(ssem, is_store=True)

    return gather_kernel


_gather = _make_gather()

_VOCAB = 1000000
_FULL_UNITS = _VOCAB // _CHUNK
_UNITS = _FULL_UNITS + 1
_UPW = (_UNITS + _NUM_WORKERS - 1) // _NUM_WORKERS


def _make_linearize():
    mesh = plsc.VectorSubcoreMesh(
        core_axis_name="c",
        subcore_axis_name="s",
        num_cores=_NUM_CORES,
        num_subcores=_NUM_SUBCORES,
    )

    @functools.partial(
        pl.kernel,
        out_type=jax.ShapeDtypeStruct((_VOCAB * _EMBED_DIM,), jnp.float32),
        mesh=mesh,
        scratch_types=[
            pltpu.VMEM((_EMBED_DIM, _CHUNK), jnp.float32),
            pltpu.VMEM((_EMBED_DIM, _CHUNK), jnp.float32),
            pltpu.VMEM((_CHUNK * _EMBED_DIM,), jnp.float32),
            pltpu.VMEM((_CHUNK * _EMBED_DIM,), jnp.float32),
            pltpu.SemaphoreType.DMA,
            pltpu.SemaphoreType.DMA,
        ],
        compiler_params=pltpu.CompilerParams(
            use_tc_tiling_on_sc=True, needs_layout_passes=False
        ),
    )
    def lin_kernel(tt_hbm, aux_hbm, out_hbm, in0_v, in1_v, ob0_v, ob1_v, gsem, ssem):
        ins = (in0_v, in1_v)
        obs = (ob0_v, ob1_v)
        wid = lax.axis_index("s") * _NUM_CORES + lax.axis_index("c")
        col_iota = lax.iota(jnp.int32, 16)
        rot = [jnp.bitwise_and(col_iota + k, 15) for k in range(16)]

        def unit_id(i):
            return i * _NUM_WORKERS + wid

        def fire_gather(i, gp):
            u = unit_id(i)

            @pl.when(u < _FULL_UNITS)
            def _():
                pltpu.async_copy(
                    tt_hbm.at[:, pl.ds(u * _CHUNK, _CHUNK)], ins[gp], gsem
                )

            @pl.when(u == _FULL_UNITS)
            def _():
                pltpu.async_copy(aux_hbm, ins[gp], gsem)

        fbase = [rot[k] * _EMBED_DIM + col_iota for k in range(16)]

        def transpose_unit(gp):
            src = ins[gp]
            dst = obs[gp]

            def jbody(ci, car):
                j0 = ci * 16
                for i0 in range(0, _EMBED_DIM, 16):
                    for k0 in range(0, 16, 8):
                        c_idx = [rot[k0 + k] + j0 for k in range(8)]
                        vals = [
                            plsc.load_gather(src, [i0 + col_iota, c_idx[k]])
                            for k in range(8)
                        ]
                        for k in range(8):
                            plsc.store_scatter(
                                dst,
                                [fbase[k0 + k] + (j0 * _EMBED_DIM + i0)],
                                vals[k],
                            )
                return car

            lax.fori_loop(0, _CHUNK // 16, jbody, 0, unroll=False)

        fire_gather(0, 0)

        def half_body(i, gp):
            u = unit_id(i)
            valid = u < _UNITS

            @pl.when(valid)
            def _():
                pltpu.make_async_copy(
                    tt_hbm.at[:, pl.ds(0, _CHUNK)], ins[0], gsem
                ).wait()

            @pl.when(unit_id(i + 1) < _UNITS)
            def _():
                fire_gather(i + 1, 1 - gp)

            @pl.when((i >= 1) & (unit_id(i - 1) < _UNITS))
            def _():
                pltpu.make_async_copy(
                    obs[0], out_hbm.at[pl.ds(0, _CHUNK * _EMBED_DIM)], ssem
                ).wait()

            @pl.when(valid)
            def _():
                transpose_unit(gp)
                f0 = jnp.where(
                    u == _FULL_UNITS,
                    (_VOCAB - _CHUNK) * _EMBED_DIM,
                    u * _CHUNK * _EMBED_DIM,
                )
                pltpu.async_copy(
                    obs[gp], out_hbm.at[pl.ds(f0, _CHUNK * _EMBED_DIM)], ssem
                )

        def body(t, carry):
            half_body(2 * t, 0)
            half_body(2 * t + 1, 1)
            return carry

        lax.fori_loop(0, _UPW // 2, body, 0, unroll=False)
        half_body(_UPW - 1, 0)

        @pl.when(unit_id(_UPW - 1) < _UNITS)
        def _():
            pltpu.make_async_copy(
                obs[0], out_hbm.at[pl.ds(0, _CHUNK * _EMBED_DIM)], ssem
            ).wait()

    return lin_kernel


_linearize = _make_linearize()


def kernel(words, emb_weight):
    idx = words.T.reshape(_NUM_WORKERS, _NUM_CHUNKS, _CHUNK).astype(jnp.int32)
    tt = emb_weight.T
    aux = lax.slice(tt, (0, _VOCAB - _CHUNK), (_EMBED_DIM, _VOCAB))
    table = _linearize(tt, aux).reshape(_VOCAB, _EMBED_DIM)
    out5 = _gather(idx, table).reshape(_HIST, _EMBED_DIM // 8, _BT, 8, _CHUNK)
    return out5.transpose(2, 4, 0, 1, 3).reshape(_BATCH, _HIST, _EMBED_DIM)
